# Initial kernel scaffold; baseline (speedup 1.0000x reference)
import jax
import jax.numpy as jnp
from jax.experimental import pallas as pl


def kernel(point1, point2):
    raise NotImplementedError("write your pallas kernel here")


# speedup vs baseline: 24.9072x; 24.9072x over previous
import jax
import jax.numpy as jnp
from jax.experimental import pallas as pl

_K = 10
_BN = 512


def _body(p1r_ref, p1_ref, p1t_ref, p2r_ref, p2_ref, out_ref):
    i = pl.program_id(1)
    n = p1_ref.shape[1]
    rows1 = p1r_ref[0]
    rows2 = p2r_ref[0]
    p1t = p1t_ref[0]

    d2all = jnp.sum(p1t * p1t, axis=0, keepdims=True)
    d2row = jnp.sum(rows1 * rows1, axis=1, keepdims=True)
    cross = jax.lax.dot_general(
        rows1.astype(jnp.bfloat16), p1t.astype(jnp.bfloat16),
        (((1,), (0,)), ((), ())),
        preferred_element_type=jnp.float32)
    dist = d2row + d2all - 2.0 * cross

    inf = jnp.float32(jnp.inf)
    d = dist
    thr = None
    v1 = None
    for t in range(_K + 1):
        thr = jnp.min(d, axis=1, keepdims=True)
        if t == 0:
            v1 = thr
        d = jnp.where(d <= thr, inf, d)

    col = jax.lax.broadcasted_iota(jnp.int32, (_BN, n), 1)
    c0 = jnp.min(jnp.where(dist == v1, col, n), axis=1, keepdims=True)
    mask = ((dist <= thr) & (col != c0)).astype(jnp.float32)
    cnt = jnp.sum(mask, axis=1, keepdims=True)
    s1 = jax.lax.dot_general(
        mask, p1_ref[0], (((1,), (0,)), ((), ())),
        preferred_element_type=jnp.float32,
        precision=jax.lax.Precision.HIGHEST)
    s2 = jax.lax.dot_general(
        mask, p2_ref[0], (((1,), (0,)), ((), ())),
        preferred_element_type=jnp.float32,
        precision=jax.lax.Precision.HIGHEST)

    diff = (s1 - s2) / cnt - (rows1 - rows2)
    part = jnp.sum(jnp.abs(diff), axis=(0, 1), keepdims=True)

    @pl.when((pl.program_id(0) == 0) & (i == 0))
    def _init():
        out_ref[...] = jnp.zeros((1, 1), jnp.float32)

    out_ref[...] += part


def kernel(point1, point2):
    b, n, d = point1.shape
    p1t = jnp.transpose(point1, (0, 2, 1))
    out = pl.pallas_call(
        _body,
        grid=(b, n // _BN),
        in_specs=[
            pl.BlockSpec((1, _BN, d), lambda bb, ii: (bb, ii, 0)),
            pl.BlockSpec((1, n, d), lambda bb, ii: (bb, 0, 0)),
            pl.BlockSpec((1, d, n), lambda bb, ii: (bb, 0, 0)),
            pl.BlockSpec((1, _BN, d), lambda bb, ii: (bb, ii, 0)),
            pl.BlockSpec((1, n, d), lambda bb, ii: (bb, 0, 0)),
        ],
        out_specs=pl.BlockSpec((1, 1), lambda bb, ii: (0, 0)),
        out_shape=jax.ShapeDtypeStruct((1, 1), jnp.float32),
    )(point1, point1, p1t, point2, point2)
    return out[0, 0] / jnp.float32(b * n * d)
